# Initial kernel scaffold; baseline (speedup 1.0000x reference)
import jax
import jax.numpy as jnp
from jax.experimental import pallas as pl


def kernel(x, inter_senders_0, inter_receivers_0, inter_senders_1, inter_receivers_1, inter_edges_0, inter_edges_1, inner_senders_0, inner_receivers_0, inner_senders_1, inner_receivers_1, inner_senders_2, inner_receivers_2, inner_edges_0, inner_edges_1, inner_edges_2, params):
    raise NotImplementedError("write your pallas kernel here")


# speedup vs baseline: 1.6202x; 1.6202x over previous
import functools

import jax
import jax.numpy as jnp
from jax import lax
from jax.experimental import pallas as pl
from jax.experimental.pallas import tpu as pltpu
from jax.experimental.pallas import tpu_sc as plsc

_N = 5376
_W = 64
_NC = 2
_NS = 16
_NW = _NC * _NS
_ROWS = _N // _NS
_SWAP = (2, 3, 0, 1, 6, 4)
_HI = jax.lax.Precision.HIGHEST


def _sc_gather(table, idx):
    e = idx.shape[0]
    epw = e // _NW
    ch = min(128, epw)
    nch = epw // ch
    mesh = plsc.VectorSubcoreMesh(core_axis_name="c", subcore_axis_name="s")

    @functools.partial(
        pl.kernel,
        mesh=mesh,
        out_type=jax.ShapeDtypeStruct((e, _W), jnp.float32),
        compiler_params=pltpu.CompilerParams(use_tc_tiling_on_sc=False),
        scratch_types=[
            pltpu.VMEM((epw,), jnp.int32),
            pltpu.VMEM((epw, _W), jnp.float32),
            pltpu.SemaphoreType.DMA,
        ],
    )
    def k(table_hbm, idx_hbm, out_hbm, idx_v, rows_v, sem):
        wid = lax.axis_index("s") * _NC + lax.axis_index("c")
        base = wid * epw
        pltpu.sync_copy(idx_hbm.at[pl.ds(base, epw)], idx_v)
        cps = [
            pltpu.async_copy(
                table_hbm.at[idx_v.at[pl.ds(j * ch, ch)]],
                rows_v.at[pl.ds(j * ch, ch)],
                sem,
            )
            for j in range(nch)
        ]
        for cp in cps:
            cp.wait()
        pltpu.sync_copy(rows_v, out_hbm.at[pl.ds(base, epw)])

    return k(table, idx)


def _sc_scatter(msgs, recv, h_prev, zeros):
    e = recv.shape[0]
    epw = e // _NW
    ch = min(128, epw)
    nch = epw // ch
    m3 = msgs.reshape(e // ch, ch, _W)
    r2 = recv.reshape(e // ch, ch)
    mesh = plsc.VectorSubcoreMesh(core_axis_name="c", subcore_axis_name="s")

    @functools.partial(
        pl.kernel,
        mesh=mesh,
        out_type=jax.ShapeDtypeStruct((_NC, _N, _W), jnp.float32),
        compiler_params=pltpu.CompilerParams(use_tc_tiling_on_sc=False),
        scratch_types=[
            pltpu.VMEM((nch, ch), jnp.int32),
            pltpu.VMEM((nch, ch, _W), jnp.float32),
            pltpu.VMEM_SHARED((_N, _W), jnp.float32),
        ],
    )
    def k(m_hbm, r_hbm, h_hbm, z_hbm, out_hbm, idx_v, vals_v, acc):
        c = lax.axis_index("c")
        s = lax.axis_index("s")
        wid = s * _NC + c
        rs = pl.ds(s * _ROWS, _ROWS)

        @pl.when(c == 0)
        def _():
            pltpu.sync_copy(h_hbm.at[rs], acc.at[rs])

        @pl.when(c == 1)
        def _():
            pltpu.sync_copy(z_hbm.at[rs], acc.at[rs])

        plsc.subcore_barrier()
        pltpu.sync_copy(r_hbm.at[pl.ds(wid * nch, nch)], idx_v)
        pltpu.sync_copy(m_hbm.at[pl.ds(wid * nch, nch)], vals_v)
        for j in range(nch):
            pltpu.sync_copy(vals_v.at[j], acc.at[idx_v.at[j]], add=True)
        plsc.subcore_barrier()
        pltpu.sync_copy(acc.at[rs], out_hbm.at[c, rs])

    return k(m3, r2, h_prev, zeros)


def _dense_msgs(attr8t, xj, hid_t, w3t, b3t, te=256):
    e = xj.shape[0]
    n_hid = len(hid_t)
    flat_w = [w for pair in hid_t for w in pair] + [w3t, b3t]

    def body(*refs):
        attr_ref, xj_ref, eye_te_ref, eye_w_ref = refs[:4]
        wrefs = refs[4:-1]
        out_ref = refs[-1]
        a = attr_ref[...]
        i = 0
        for _ in range(n_hid):
            a = jnp.dot(wrefs[i][...], a, preferred_element_type=jnp.float32)
            a = jnp.maximum(a + wrefs[i + 1][...], 0.0)
            i += 2
        w3t_ref, b3t_ref = wrefs[i], wrefs[i + 1]
        wt = (jnp.dot(w3t_ref[...], a, preferred_element_type=jnp.float32)
              + b3t_ref[...])
        xjt = jax.lax.dot_general(
            xj_ref[...], eye_te_ref[...], (((0,), (0,)), ((), ())),
            preferred_element_type=jnp.float32, precision=_HI)
        xjt = xjt.astype(jnp.bfloat16).astype(jnp.float32)
        wt3 = wt.astype(jnp.bfloat16).astype(jnp.float32).reshape(_W, _W, te)
        msgst = jnp.sum(
            jax.lax.broadcast_in_dim(xjt, (_W, _W, te), (0, 2)) * wt3, axis=0)
        out_ref[...] = jax.lax.dot_general(
            msgst, eye_w_ref[...], (((0,), (0,)), ((), ())),
            preferred_element_type=jnp.float32, precision=_HI)

    in_specs = [
        pl.BlockSpec((8, te), lambda i: (0, i)),
        pl.BlockSpec((te, _W), lambda i: (i, 0)),
        pl.BlockSpec((te, te), lambda i: (0, 0)),
        pl.BlockSpec((_W, _W), lambda i: (0, 0)),
    ] + [pl.BlockSpec(w.shape, lambda i: (0, 0)) for w in flat_w]
    return pl.pallas_call(
        body,
        grid=(e // te,),
        in_specs=in_specs,
        out_specs=pl.BlockSpec((te, _W), lambda i: (i, 0)),
        out_shape=jax.ShapeDtypeStruct((e, _W), jnp.float32),
    )(attr8t, xj, jnp.eye(te, dtype=jnp.float32), jnp.eye(_W, dtype=jnp.float32), *flat_w)


def _first_mlp(x8, w, b):
    def body(x_ref, w_ref, b_ref, o_ref):
        o_ref[...] = (
            jnp.dot(x_ref[...], w_ref[...], preferred_element_type=jnp.float32)
            + b_ref[...]
        )

    return pl.pallas_call(
        body, out_shape=jax.ShapeDtypeStruct((_N, _W), jnp.float32)
    )(x8, w, b)


def _finish(parts):
    def body(p_ref, o_ref):
        o_ref[...] = jnp.maximum(p_ref[0] + p_ref[1], 0.0)

    return pl.pallas_call(
        body, out_shape=jax.ShapeDtypeStruct((_N, _W), jnp.float32)
    )(parts)


def _final_mlp(h4k, w4, b4, w5, b5):
    def body(h_ref, w4_ref, b4_ref, w5_ref, b5_ref, o_ref):
        t = jnp.dot(h_ref[...], w4_ref[...], preferred_element_type=jnp.float32)
        t = jnp.maximum(t + b4_ref[...], 0.0)
        o_ref[...] = (
            jnp.dot(t, w5_ref[...], preferred_element_type=jnp.float32)
            + b5_ref[...]
        )

    return pl.pallas_call(
        body, out_shape=jax.ShapeDtypeStruct((h4k.shape[0], 1), jnp.float32)
    )(h4k, w4, b4, w5, b5)


def _pad8(a):
    return jnp.pad(a, ((0, 0), (0, 8 - a.shape[1])))


def kernel(x, inter_senders_0, inter_receivers_0, inter_senders_1, inter_receivers_1, inter_edges_0, inter_edges_1, inner_senders_0, inner_receivers_0, inner_senders_1, inner_receivers_1, inner_senders_2, inner_receivers_2, inner_edges_0, inner_edges_1, inner_edges_2, params):
    zeros = jnp.zeros((_N, _W), jnp.float32)

    def prep(name):
        layers = params[name]
        hid_t = []
        for w, b in layers[:-1]:
            if w.shape[0] < 8:
                w = jnp.pad(w, ((0, 8 - w.shape[0]), (0, 0)))
            hid_t.append((w.T, b.reshape(-1, 1)))
        w3, b3 = layers[-1]
        return hid_t, w3.T, b3.reshape(-1, 1)

    def conv(h, snd, rcv, attr, name, swap=False):
        if swap:
            attr = attr[:, list(_SWAP)]
        hid_t, w3t, b3t = prep(name)
        xj = _sc_gather(h, snd)
        msgs = _dense_msgs(_pad8(attr).T, xj, hid_t, w3t, b3t)
        parts = _sc_scatter(msgs, rcv, h, zeros)
        return _finish(parts)

    (w0, b0), = params['first']
    h = _first_mlp(_pad8(x), jnp.pad(w0, ((0, 2), (0, 0))), b0.reshape(1, -1))
    h = conv(h, inter_senders_0, inter_receivers_0, inter_edges_0, 'K12')
    h = conv(h, inter_senders_1, inter_receivers_1, inter_edges_1, 'K23')
    h = conv(h, inner_senders_2, inner_receivers_2, inner_edges_2, 'K33')
    h = conv(h, inner_senders_1, inner_receivers_1, inner_edges_1, 'K22')
    h = conv(h, inter_receivers_1, inter_senders_1, inter_edges_1, 'K32', swap=True)
    h = conv(h, inner_senders_0, inner_receivers_0, inner_edges_0, 'K11')
    h = conv(h, inter_receivers_0, inter_senders_0, inter_edges_0, 'K21', swap=True)
    (w4, b4), (w5, b5) = params['final']
    return _final_mlp(h[:4096], w4, b4.reshape(1, -1), w5, b5.reshape(1, -1))
